# Initial kernel scaffold; baseline (speedup 1.0000x reference)
import jax
import jax.numpy as jnp
from jax.experimental import pallas as pl


def kernel(memory_obs, memory_reward, obs, reward, i, sample_indices):
    raise NotImplementedError("write your pallas kernel here")


# speedup vs baseline: 1.0738x; 1.0738x over previous
import functools

import jax
import jax.numpy as jnp
from jax import lax
from jax.experimental import pallas as pl
from jax.experimental.pallas import tpu as pltpu
from jax.experimental.pallas import tpu_sc as plsc


def _build_sc_kernel(M, D, B, NC, NS, L):
    NW = NC * NS
    bpw = B // NW
    n_grp = bpw // L
    n_q = D // L
    mesh = plsc.VectorSubcoreMesh(core_axis_name="c", subcore_axis_name="s")

    @functools.partial(
        pl.kernel,
        out_type=(
            jax.ShapeDtypeStruct((B, D), jnp.float32),
            jax.ShapeDtypeStruct((B,), jnp.float32),
        ),
        mesh=mesh,
        compiler_params=pltpu.CompilerParams(use_tc_tiling_on_sc=False, needs_layout_passes=False),
        scratch_types=[
            pltpu.VMEM((bpw,), jnp.int32),
            pltpu.VMEM((L,), jnp.int32),
            pltpu.VMEM((bpw,), jnp.int32),
            pltpu.VMEM((bpw,), jnp.int32),
            pltpu.VMEM((bpw, D), jnp.float32),
            pltpu.VMEM((bpw, D), jnp.float32),
            pltpu.VMEM((bpw,), jnp.float32),
            pltpu.VMEM((bpw,), jnp.float32),
            pltpu.SemaphoreType.DMA,
            pltpu.SemaphoreType.DMA,
            pltpu.SemaphoreType.DMA,
            pltpu.SemaphoreType.DMA,
        ],
    )
    def k(mem_obs_h, mem_rew_h, obs_h, rew_h, sidx_h, ivec_h,
          out_obs_h, out_rew_h,
          idx_v, ivec_v, off_v, obs_idx_v, rows_v, obs_rows_v, rew_v, obs_rew_v,
          sem0, sem1, sem2, sem3):
        wid = lax.axis_index("s") * NC + lax.axis_index("c")
        base = wid * bpw
        pltpu.sync_copy(sidx_h.at[pl.ds(base, bpw)], idx_v)
        pltpu.sync_copy(ivec_h, ivec_v)
        c_mem = pltpu.async_copy(mem_obs_h.at[idx_v], rows_v, sem0)
        c_rew = pltpu.async_copy(mem_rew_h.at[idx_v], rew_v, sem1)
        iv = ivec_v[...]

        def grp(g, cnt):
            gb = pl.multiple_of(g * L, L)
            s = idx_v[pl.ds(gb, L)]
            off = s - iv
            off = jnp.where(off < 0, off + M, off)
            hit = off < B
            off_v[pl.ds(gb, L)] = off
            obs_idx_v[pl.ds(gb, L)] = jnp.where(hit, off, 0)
            return cnt + plsc.all_reduce_population_count(hit)

        nhit = lax.fori_loop(0, n_grp, grp, jnp.zeros((L,), jnp.int32))
        c_mem.wait()
        c_rew.wait()

        @pl.when(nhit[0] > 0)
        def _fixup():
            pltpu.async_copy(obs_h.at[obs_idx_v], obs_rows_v, sem2).wait()
            pltpu.async_copy(rew_h.at[obs_idx_v], obs_rew_v, sem3).wait()

            def rgrp(g, c):
                gb = pl.multiple_of(g * L, L)
                off = off_v[pl.ds(gb, L)]
                hit = off < B
                rew_v[pl.ds(gb, L)] = jnp.where(
                    hit, obs_rew_v[pl.ds(gb, L)], rew_v[pl.ds(gb, L)])
                return c

            lax.fori_loop(0, n_grp, rgrp, jnp.int32(0))

            def row_fix(g, c):
                gb = pl.multiple_of(g * L, L)
                off = off_v[pl.ds(gb, L)]
                for j in range(L):
                    @pl.when(off[j] < B)
                    def _(j=j):
                        b = gb + j
                        for q in range(n_q):
                            rows_v[b, pl.ds(q * L, L)] = (
                                obs_rows_v[b, pl.ds(q * L, L)])
                return c

            lax.fori_loop(0, n_grp, row_fix, jnp.int32(0))

        pltpu.sync_copy(rows_v, out_obs_h.at[pl.ds(base, bpw)])
        pltpu.sync_copy(rew_v, out_rew_h.at[pl.ds(base, bpw)])

    return k


def kernel(memory_obs, memory_reward, obs, reward, i, sample_indices):
    M, D = memory_obs.shape
    B = obs.shape[0]
    info = plsc.get_sparse_core_info()
    NC, NS, L = info.num_cores, info.num_subcores, info.num_lanes
    sidx = sample_indices.astype(jnp.int32)
    i_vec = jnp.full((L,), i, dtype=jnp.int32)
    k = _build_sc_kernel(M, D, B, NC, NS, L)
    out_obs, out_rew = k(memory_obs, memory_reward, obs, reward, sidx, i_vec)
    return jnp.concatenate([out_obs, out_rew[:, None]], axis=1)
